# baseline (speedup 1.0000x reference)
import jax
import jax.numpy as jnp
from jax.experimental import pallas as pl
from jax.experimental.pallas import tpu as pltpu

_B = 64
_V = 100000
_ROWS = 8
_HI_SENTINEL = 0x3F800001
_PASSES = 17


def _sampler_body(logits_ref, noise_ref, temp_ref, topp_ref, minp_ref,
                  topk_ref, probs_ref, tok_ref, slp_ref):
    x = logits_ref[...] / temp_ref[...]
    m = jnp.max(x, axis=-1, keepdims=True)
    e = jnp.exp(x - m)
    z = jnp.sum(e, axis=-1, keepdims=True)
    u = jax.lax.bitcast_convert_type(e, jnp.int32)

    k = topk_ref[...]
    mass_limit = topp_ref[...] * z
    r1 = (_ROWS, 1)
    lo_k = jnp.zeros(r1, jnp.int32)
    hi_k = jnp.full(r1, _HI_SENTINEL, jnp.int32)
    lo_p = jnp.full(r1, -1, jnp.int32)
    hi_p = jnp.full(r1, _HI_SENTINEL, jnp.int32)

    def srch(lo, hi, measure, below):
        q = jnp.maximum((hi - lo) >> 2, 1)
        mids = [lo + q * s for s in (1, 2, 3)]
        for mid in mids:
            c = below(measure(mid))
            hi = jnp.where(c, jnp.minimum(hi, mid), hi)
            lo = jnp.where(c, lo, jnp.maximum(lo, mid))
        return lo, hi

    def body(_, carry):
        lo_k, hi_k, lo_p, hi_p = carry
        lo_k, hi_k = srch(
            lo_k, hi_k,
            lambda t: jnp.sum((u >= t).astype(jnp.int32), axis=-1,
                              keepdims=True),
            lambda cnt: cnt < k)
        lo_p, hi_p = srch(
            lo_p, hi_p,
            lambda t: jnp.sum(jnp.where(u > t, e, 0.0), axis=-1,
                              keepdims=True),
            lambda mass: mass <= mass_limit)
        return lo_k, hi_k, lo_p, hi_p

    lo_k, hi_k, lo_p, hi_p = jax.lax.fori_loop(
        0, _PASSES, body, (lo_k, hi_k, lo_p, hi_p))

    thr_bits = jnp.maximum(lo_k, jnp.minimum(hi_p, 0x3F800000))
    thr = jnp.maximum(jax.lax.bitcast_convert_type(thr_bits, jnp.float32),
                      minp_ref[...])
    ez = jnp.where(e >= thr, e, 0.0)
    z2 = jnp.sum(ez, axis=-1, keepdims=True)
    probs = ez * (1.0 / z2)
    probs_ref[...] = probs

    r = probs / (-jnp.log(noise_ref[...]))
    rmax = jnp.max(r, axis=-1, keepdims=True)
    iota = jax.lax.broadcasted_iota(jnp.int32, r.shape, 1)
    idx = jnp.min(jnp.where(r == rmax, iota, _V), axis=-1, keepdims=True)
    tok_ref[...] = idx

    xs = jnp.max(jnp.where(iota == idx, x, -jnp.inf), axis=-1, keepdims=True)
    slp_ref[...] = (xs - m) - jnp.log(z2)


def kernel(logits, temperatures, top_ps, min_ps, top_ks, noise):
    nb = _B // _ROWS
    row_spec = pl.BlockSpec((_ROWS, _V), lambda i: (i, 0))
    par_spec = pl.BlockSpec((_ROWS, 1), lambda i: (i, 0))
    probs, tok, slp = pl.pallas_call(
        _sampler_body,
        grid=(nb,),
        in_specs=[row_spec, row_spec, par_spec, par_spec, par_spec, par_spec],
        out_specs=[row_spec, par_spec, par_spec],
        out_shape=[
            jax.ShapeDtypeStruct((_B, _V), jnp.float32),
            jax.ShapeDtypeStruct((_B, 1), jnp.int32),
            jax.ShapeDtypeStruct((_B, 1), jnp.float32),
        ],
        compiler_params=pltpu.CompilerParams(
            dimension_semantics=("parallel",)),
    )(logits, noise, temperatures.reshape(_B, 1), top_ps.reshape(_B, 1),
      min_ps.reshape(_B, 1), top_ks.reshape(_B, 1))
    return probs, tok.reshape(_B), slp


# speedup vs baseline: 1.0975x; 1.0975x over previous
import jax
import jax.numpy as jnp
from jax.experimental import pallas as pl
from jax.experimental.pallas import tpu as pltpu

_B = 64
_V = 100000
_ROWS = 8
_HI_SENTINEL = 0x3F800001
_PASSES = 30


def _sampler_body(logits_ref, noise_ref, temp_ref, topp_ref, minp_ref,
                  topk_ref, probs_ref, tok_ref, slp_ref):
    x = logits_ref[...] / temp_ref[...]
    m = jnp.max(x, axis=-1, keepdims=True)
    e = jnp.exp(x - m)
    z = jnp.sum(e, axis=-1, keepdims=True)

    k = topk_ref[...]
    mass_limit = topp_ref[...] * z
    r1 = (_ROWS, 1)
    lo_k = jnp.zeros(r1, jnp.int32)
    hi_k = jnp.full(r1, _HI_SENTINEL, jnp.int32)
    lo_p = jnp.full(r1, -1, jnp.int32)
    hi_p = jnp.full(r1, _HI_SENTINEL, jnp.int32)

    def body(_, carry):
        lo_k, hi_k, lo_p, hi_p = carry
        mid_k = lo_k + ((hi_k - lo_k) >> 1)
        mid_p = lo_p + ((hi_p - lo_p) >> 1)
        fmid_k = jax.lax.bitcast_convert_type(mid_k, jnp.float32)
        fmid_p = jax.lax.bitcast_convert_type(jnp.maximum(mid_p, 0),
                                              jnp.float32)
        cnt = jnp.sum((e >= fmid_k).astype(jnp.int32), axis=-1,
                      keepdims=True)
        mass = jnp.sum(jnp.where(e > fmid_p, e, 0.0), axis=-1,
                       keepdims=True)
        ck = cnt >= k
        lo_k = jnp.where(ck, mid_k, lo_k)
        hi_k = jnp.where(ck, hi_k, mid_k)
        cp = mass <= mass_limit
        hi_p = jnp.where(cp, mid_p, hi_p)
        lo_p = jnp.where(cp, lo_p, mid_p)
        return lo_k, hi_k, lo_p, hi_p

    lo_k, hi_k, lo_p, hi_p = jax.lax.fori_loop(
        0, _PASSES, body, (lo_k, hi_k, lo_p, hi_p))

    thr_bits = jnp.maximum(lo_k, jnp.minimum(hi_p, 0x3F800000))
    thr = jnp.maximum(jax.lax.bitcast_convert_type(thr_bits, jnp.float32),
                      minp_ref[...])
    ez = jnp.where(e >= thr, e, 0.0)
    z2 = jnp.sum(ez, axis=-1, keepdims=True)
    probs = ez * (1.0 / z2)
    probs_ref[...] = probs

    r = probs / (-jnp.log(noise_ref[...]))
    rmax = jnp.max(r, axis=-1, keepdims=True)
    iota = jax.lax.broadcasted_iota(jnp.int32, r.shape, 1)
    idx = jnp.min(jnp.where(r == rmax, iota, _V), axis=-1, keepdims=True)
    tok_ref[...] = idx

    xs = jnp.max(jnp.where(iota == idx, x, -jnp.inf), axis=-1, keepdims=True)
    slp_ref[...] = (xs - m) - jnp.log(z2)


def kernel(logits, temperatures, top_ps, min_ps, top_ks, noise):
    nb = _B // _ROWS
    row_spec = pl.BlockSpec((_ROWS, _V), lambda i: (i, 0))
    par_spec = pl.BlockSpec((_ROWS, 1), lambda i: (i, 0))
    probs, tok, slp = pl.pallas_call(
        _sampler_body,
        grid=(nb,),
        in_specs=[row_spec, row_spec, par_spec, par_spec, par_spec, par_spec],
        out_specs=[row_spec, par_spec, par_spec],
        out_shape=[
            jax.ShapeDtypeStruct((_B, _V), jnp.float32),
            jax.ShapeDtypeStruct((_B, 1), jnp.int32),
            jax.ShapeDtypeStruct((_B, 1), jnp.float32),
        ],
        compiler_params=pltpu.CompilerParams(
            dimension_semantics=("parallel",)),
    )(logits, noise, temperatures.reshape(_B, 1), top_ps.reshape(_B, 1),
      min_ps.reshape(_B, 1), top_ks.reshape(_B, 1))
    return probs, tok.reshape(_B), slp


# speedup vs baseline: 1.4733x; 1.3424x over previous
import jax
import jax.numpy as jnp
from jax.experimental import pallas as pl
from jax.experimental.pallas import tpu as pltpu

_B = 64
_V = 100000
_ROWS = 16
_HI_SENTINEL = 0x3F800001
_PASSES = 30


def _sampler_body(logits_ref, noise_ref, temp_ref, topp_ref, minp_ref,
                  topk_ref, probs_ref, tok_ref, slp_ref):
    x = logits_ref[...] / temp_ref[...]
    m = jnp.max(x, axis=-1, keepdims=True)
    e = jnp.exp(x - m)
    z = jnp.sum(e, axis=-1, keepdims=True)

    k = topk_ref[...]
    mass_limit = topp_ref[...] * z
    r1 = (_ROWS, 1)
    lo_k = jnp.zeros(r1, jnp.int32)
    hi_k = jnp.full(r1, _HI_SENTINEL, jnp.int32)
    lo_p = jnp.full(r1, -1, jnp.int32)
    hi_p = jnp.full(r1, _HI_SENTINEL, jnp.int32)

    def body(_, carry):
        lo_k, hi_k, lo_p, hi_p = carry
        mid_k = lo_k + ((hi_k - lo_k) >> 1)
        mid_p = lo_p + ((hi_p - lo_p) >> 1)
        fmid_k = jax.lax.bitcast_convert_type(mid_k, jnp.float32)
        fmid_p = jax.lax.bitcast_convert_type(jnp.maximum(mid_p, 0),
                                              jnp.float32)
        cnt = jnp.sum((e >= fmid_k).astype(jnp.int32), axis=-1,
                      keepdims=True)
        mass = jnp.sum(jnp.where(e > fmid_p, e, 0.0), axis=-1,
                       keepdims=True)
        ck = cnt >= k
        lo_k = jnp.where(ck, mid_k, lo_k)
        hi_k = jnp.where(ck, hi_k, mid_k)
        cp = mass <= mass_limit
        hi_p = jnp.where(cp, mid_p, hi_p)
        lo_p = jnp.where(cp, lo_p, mid_p)
        return lo_k, hi_k, lo_p, hi_p

    lo_k, hi_k, lo_p, hi_p = jax.lax.fori_loop(
        0, _PASSES, body, (lo_k, hi_k, lo_p, hi_p))

    thr_bits = jnp.maximum(lo_k, jnp.minimum(hi_p, 0x3F800000))
    thr = jnp.maximum(jax.lax.bitcast_convert_type(thr_bits, jnp.float32),
                      minp_ref[...])
    ez = jnp.where(e >= thr, e, 0.0)
    z2 = jnp.sum(ez, axis=-1, keepdims=True)
    probs = ez * (1.0 / z2)
    probs_ref[...] = probs

    r = probs / (-jnp.log(noise_ref[...]))
    rmax = jnp.max(r, axis=-1, keepdims=True)
    iota = jax.lax.broadcasted_iota(jnp.int32, r.shape, 1)
    idx = jnp.min(jnp.where(r == rmax, iota, _V), axis=-1, keepdims=True)
    tok_ref[...] = idx

    xs = jnp.max(jnp.where(iota == idx, x, -jnp.inf), axis=-1, keepdims=True)
    slp_ref[...] = (xs - m) - jnp.log(z2)


def kernel(logits, temperatures, top_ps, min_ps, top_ks, noise):
    nb = _B // _ROWS
    row_spec = pl.BlockSpec((_ROWS, _V), lambda i: (i, 0))
    par_spec = pl.BlockSpec((_ROWS, 1), lambda i: (i, 0))
    probs, tok, slp = pl.pallas_call(
        _sampler_body,
        grid=(nb,),
        in_specs=[row_spec, row_spec, par_spec, par_spec, par_spec, par_spec],
        out_specs=[row_spec, par_spec, par_spec],
        out_shape=[
            jax.ShapeDtypeStruct((_B, _V), jnp.float32),
            jax.ShapeDtypeStruct((_B, 1), jnp.int32),
            jax.ShapeDtypeStruct((_B, 1), jnp.float32),
        ],
        compiler_params=pltpu.CompilerParams(
            dimension_semantics=("parallel",)),
    )(logits, noise, temperatures.reshape(_B, 1), top_ps.reshape(_B, 1),
      min_ps.reshape(_B, 1), top_ks.reshape(_B, 1))
    return probs, tok.reshape(_B), slp


# speedup vs baseline: 1.4738x; 1.0004x over previous
import jax
import jax.numpy as jnp
from jax.experimental import pallas as pl
from jax.experimental.pallas import tpu as pltpu

_B = 64
_V = 100000
_ROWS = 16
_HI_SENTINEL = 0x3F800001


def _sampler_body(logits_ref, noise_ref, temp_ref, topp_ref, minp_ref,
                  topk_ref, probs_ref, tok_ref, slp_ref):
    x = logits_ref[...] / temp_ref[...]
    m = jnp.max(x, axis=-1, keepdims=True)
    e = jnp.exp(x - m)
    z = jnp.sum(e, axis=-1, keepdims=True)

    k = topk_ref[...]
    mass_limit = topp_ref[...] * z
    r1 = (_ROWS, 1)
    emin_bits = jax.lax.bitcast_convert_type(
        jnp.min(e, axis=-1, keepdims=True), jnp.int32)
    lo_k = jnp.maximum(emin_bits - 1, 0)
    hi_k = jnp.full(r1, _HI_SENTINEL, jnp.int32)
    lo_p = emin_bits - 1
    hi_p = jnp.full(r1, _HI_SENTINEL, jnp.int32)

    def cond(carry):
        lo_k, hi_k, lo_p, hi_p = carry
        return jnp.any((hi_k - lo_k > 1) | (hi_p - lo_p > 1))

    def body(carry):
        lo_k, hi_k, lo_p, hi_p = carry
        mid_k = lo_k + ((hi_k - lo_k) >> 1)
        mid_p = lo_p + ((hi_p - lo_p) >> 1)
        fmid_k = jax.lax.bitcast_convert_type(mid_k, jnp.float32)
        fmid_p = jax.lax.bitcast_convert_type(jnp.maximum(mid_p, 0),
                                              jnp.float32)
        cnt = jnp.sum((e >= fmid_k).astype(jnp.int32), axis=-1,
                      keepdims=True)
        mass = jnp.sum(jnp.where(e > fmid_p, e, 0.0), axis=-1,
                       keepdims=True)
        ck = cnt >= k
        lo_k = jnp.where(ck, mid_k, lo_k)
        hi_k = jnp.where(ck, hi_k, mid_k)
        cp = mass <= mass_limit
        hi_p = jnp.where(cp, mid_p, hi_p)
        lo_p = jnp.where(cp, lo_p, mid_p)
        return lo_k, hi_k, lo_p, hi_p

    lo_k, hi_k, lo_p, hi_p = jax.lax.while_loop(
        cond, body, (lo_k, hi_k, lo_p, hi_p))

    thr_bits = jnp.maximum(lo_k, jnp.minimum(hi_p, 0x3F800000))
    thr = jnp.maximum(jax.lax.bitcast_convert_type(thr_bits, jnp.float32),
                      minp_ref[...])
    ez = jnp.where(e >= thr, e, 0.0)
    z2 = jnp.sum(ez, axis=-1, keepdims=True)
    probs = ez * (1.0 / z2)
    probs_ref[...] = probs

    r = probs / (-jnp.log(noise_ref[...]))
    rmax = jnp.max(r, axis=-1, keepdims=True)
    iota = jax.lax.broadcasted_iota(jnp.int32, r.shape, 1)
    idx = jnp.min(jnp.where(r == rmax, iota, _V), axis=-1, keepdims=True)
    tok_ref[...] = idx

    xs = jnp.max(jnp.where(iota == idx, x, -jnp.inf), axis=-1, keepdims=True)
    slp_ref[...] = (xs - m) - jnp.log(z2)


def kernel(logits, temperatures, top_ps, min_ps, top_ks, noise):
    nb = _B // _ROWS
    row_spec = pl.BlockSpec((_ROWS, _V), lambda i: (i, 0))
    par_spec = pl.BlockSpec((_ROWS, 1), lambda i: (i, 0))
    probs, tok, slp = pl.pallas_call(
        _sampler_body,
        grid=(nb,),
        in_specs=[row_spec, row_spec, par_spec, par_spec, par_spec, par_spec],
        out_specs=[row_spec, par_spec, par_spec],
        out_shape=[
            jax.ShapeDtypeStruct((_B, _V), jnp.float32),
            jax.ShapeDtypeStruct((_B, 1), jnp.int32),
            jax.ShapeDtypeStruct((_B, 1), jnp.float32),
        ],
        compiler_params=pltpu.CompilerParams(
            dimension_semantics=("parallel",)),
    )(logits, noise, temperatures.reshape(_B, 1), top_ps.reshape(_B, 1),
      min_ps.reshape(_B, 1), top_ks.reshape(_B, 1))
    return probs, tok.reshape(_B), slp
